# baseline (speedup 1.0000x reference)
import jax
import jax.numpy as jnp
from jax import lax
from jax.experimental import pallas as pl
from jax.experimental.pallas import tpu as pltpu
from jax.experimental.pallas import tpu_sc as plsc

N = 10000
E = 320000
D = 128
G = 64

NC = 2
NS = 16
NW = NC * NS
K = 64
CH = 160
NPASS = 10
CHP = CH // NPASS
EPW = CH * K
EP = NW * EPW
NB = 4
NP_ = 10240
ROWS = NP_ // NS

BN = 1000
GRID = N // BN


def _leaky(x):
    return jnp.where(x > 0, x, 0.01 * x)


def _fc1_body(x_ref, w_ref, b_ref, o_ref):
    o_ref[...] = lax.dot_general(
        x_ref[...], w_ref[...], (((1,), (1,)), ((), ())),
        preferred_element_type=jnp.float32) + b_ref[...]


def _fc1(x, W, b):
    return pl.pallas_call(
        _fc1_body,
        grid=(GRID,),
        in_specs=[
            pl.BlockSpec((BN, D), lambda i: (i, 0)),
            pl.BlockSpec((D, D), lambda i: (0, 0)),
            pl.BlockSpec((1, D), lambda i: (0, 0)),
        ],
        out_specs=pl.BlockSpec((BN, D), lambda i: (i, 0)),
        out_shape=jax.ShapeDtypeStruct((N, D), jnp.float32),
    )(x, W, b.reshape(1, D))


def _conv_linear_body(a_ref, d_ref, w1_ref, b1_ref, w_ref, b_ref, o_ref):
    a = a_ref[0] + a_ref[1]
    deg = jnp.clip(d_ref[...][:, 0] + d_ref[...][:, 1], 1.0, None)
    m = a / deg[:, None]
    m = lax.dot_general(
        m, w1_ref[...], (((1,), (1,)), ((), ())),
        preferred_element_type=jnp.float32) + b1_ref[...]
    o_ref[...] = _leaky(lax.dot_general(
        m, w_ref[...], (((1,), (1,)), ((), ())),
        preferred_element_type=jnp.float32) + b_ref[...])


def _conv_linear(aggp, degp, W1, b1, W, b):
    return pl.pallas_call(
        _conv_linear_body,
        grid=(GRID,),
        in_specs=[
            pl.BlockSpec((NC, BN, D), lambda i: (0, i, 0)),
            pl.BlockSpec((BN, NC), lambda i: (i, 0)),
            pl.BlockSpec((D, D), lambda i: (0, 0)),
            pl.BlockSpec((1, D), lambda i: (0, 0)),
            pl.BlockSpec((D, D), lambda i: (0, 0)),
            pl.BlockSpec((1, D), lambda i: (0, 0)),
        ],
        out_specs=pl.BlockSpec((BN, D), lambda i: (i, 0)),
        out_shape=jax.ShapeDtypeStruct((N, D), jnp.float32),
    )(aggp, degp, W1, b1.reshape(1, D), W, b.reshape(1, D))


def _tail_body(a_ref, d_ref, bat_ref, wg_ref, bg_ref, w2_ref, b2_ref,
               w3_ref, b3_ref, o_ref, sums_sc, cnt_sc):
    i = pl.program_id(0)

    @pl.when(i == 0)
    def _():
        sums_sc[...] = jnp.zeros_like(sums_sc)
        cnt_sc[...] = jnp.zeros_like(cnt_sc)

    a = a_ref[0] + a_ref[1]
    deg = jnp.clip(d_ref[...][:, 0] + d_ref[...][:, 1], 1.0, None)
    m = a / deg[:, None]
    h2 = _leaky(lax.dot_general(
        m, wg_ref[...], (((1,), (1,)), ((), ())),
        preferred_element_type=jnp.float32) + bg_ref[...])

    bvec = bat_ref[...][:, 0]
    onehot = (bvec[:, None] ==
              lax.broadcasted_iota(jnp.int32, (1, G), 1)).astype(jnp.float32)
    sums_sc[...] += lax.dot_general(
        onehot, h2, (((0,), (0,)), ((), ())),
        preferred_element_type=jnp.float32)
    cnt_sc[...] += jnp.sum(onehot, axis=0)[:, None]

    @pl.when(i == GRID - 1)
    def _():
        cnt = jnp.clip(cnt_sc[...][:, 0:1], 1.0, None)
        hg = sums_sc[...] / cnt
        nrm = jnp.clip(
            jnp.sqrt(jnp.sum(hg * hg, axis=1, keepdims=True)), 1e-12, None)
        hg = hg / nrm
        h = _leaky(lax.dot_general(
            hg, w2_ref[...], (((1,), (1,)), ((), ())),
            preferred_element_type=jnp.float32) + b2_ref[...])
        o_ref[...] = lax.dot_general(
            h, w3_ref[...], (((1,), (1,)), ((), ())),
            preferred_element_type=jnp.float32) + b3_ref[...]


def _tail(aggp, degp, batch, Wg, bg, W2, b2, W3, b3):
    return pl.pallas_call(
        _tail_body,
        grid=(GRID,),
        in_specs=[
            pl.BlockSpec((NC, BN, D), lambda i: (0, i, 0)),
            pl.BlockSpec((BN, NC), lambda i: (i, 0)),
            pl.BlockSpec((BN, 1), lambda i: (i, 0)),
            pl.BlockSpec((D, D), lambda i: (0, 0)),
            pl.BlockSpec((1, D), lambda i: (0, 0)),
            pl.BlockSpec((D, D), lambda i: (0, 0)),
            pl.BlockSpec((1, D), lambda i: (0, 0)),
            pl.BlockSpec((D, D), lambda i: (0, 0)),
            pl.BlockSpec((1, D), lambda i: (0, 0)),
        ],
        out_specs=pl.BlockSpec((G, D), lambda i: (0, 0)),
        out_shape=jax.ShapeDtypeStruct((G, D), jnp.float32),
        scratch_shapes=[
            pltpu.VMEM((G, D), jnp.float32),
            pltpu.VMEM((G, D), jnp.float32),
        ],
    )(aggp, degp, batch.reshape(N, 1), Wg, bg.reshape(1, D),
      W2, b2.reshape(1, D), W3, b3.reshape(1, D))


_MESH = dict(core_axis_name="c", subcore_axis_name="s",
             num_cores=NC, num_subcores=NS)


def _sc_agg(h, ei_r, zrows, zdeg, with_deg):
    out_type = [jax.ShapeDtypeStruct((NC, NP_, D), jnp.float32)]
    scratch = (
        [pltpu.VMEM((CHP, K), jnp.int32),
         pltpu.VMEM((CHP, K), jnp.int32)]
        + [pltpu.VMEM((K, D), jnp.float32)] * NB
        + [pltpu.VMEM_SHARED((NP_, D), jnp.float32)]
        + [pltpu.SemaphoreType.DMA] * (2 * NB)
    )
    if with_deg:
        out_type.append(jax.ShapeDtypeStruct((NC, NP_), jnp.float32))
        scratch += [
            pltpu.VMEM((K,), jnp.float32),
            pltpu.VMEM_SHARED((NP_,), jnp.float32),
            pltpu.SemaphoreType.DMA,
        ]

    def body(h_hbm, ei_hbm, zr_hbm, zd_hbm, *rest):
        agg_out = rest[0]
        rest = rest[1:]
        if with_deg:
            deg_out = rest[0]
            rest = rest[1:]
        src_v, dst_v = rest[0], rest[1]
        msgs = rest[2:2 + NB]
        agg_sh = rest[2 + NB]
        sem_g = rest[3 + NB:3 + 2 * NB]
        sem_s = rest[3 + 2 * NB:3 + 3 * NB]
        if with_deg:
            ones_v, deg_sh, sem_d = rest[3 + 3 * NB:]
        cid = lax.axis_index("c")
        sid = lax.axis_index("s")
        wid = cid * NS + sid

        pltpu.sync_copy(zr_hbm, agg_sh.at[pl.ds(sid * ROWS, ROWS)])
        if with_deg:
            @pl.when(sid == 0)
            def _():
                pltpu.sync_copy(zd_hbm, deg_sh)
            for j in range(K // 16):
                ones_v[pl.ds(j * 16, 16)] = jnp.ones((16,), jnp.float32)

        plsc.subcore_barrier()

        def wait_gather(b):
            pltpu.make_async_copy(h_hbm.at[pl.ds(0, K)], msgs[b],
                                  sem_g[b]).wait()

        def wait_scatter(b):
            pltpu.make_async_copy(h_hbm.at[pl.ds(0, K)], msgs[b],
                                  sem_s[b]).wait()

        def wait_deg():
            pltpu.make_async_copy(zd_hbm.at[pl.ds(0, K)], ones_v,
                                  sem_d).wait()

        def fire_gather(c, b):
            pltpu.async_copy(h_hbm.at[src_v.at[c]], msgs[b], sem_g[b])

        def fire_scatter(c, b):
            pltpu.async_copy(msgs[b], agg_sh.at[dst_v.at[c]], sem_s[b],
                             add=True)

        def fire_deg(c):
            if with_deg:
                pltpu.async_copy(ones_v, deg_sh.at[dst_v.at[c]], sem_d,
                                 add=True)

        def dwait():
            if with_deg:
                wait_deg()

        def step(c, b, first=False, fire=True):
            if not first:
                wait_scatter((b + NB - 1) % NB)
                dwait()
            if fire:
                fire_gather(c + NB - 1, (b + NB - 1) % NB)
            wait_gather(b)
            fire_scatter(c, b)
            fire_deg(c)

        for p in range(NPASS):
            pltpu.sync_copy(ei_hbm.at[0, wid, pl.ds(p * CHP, CHP)], src_v)
            pltpu.sync_copy(ei_hbm.at[1, wid, pl.ds(p * CHP, CHP)], dst_v)
            for b in range(NB - 1):
                fire_gather(b, b)
            for b in range(NB):
                step(b, b, first=(b == 0))

            def outer(i, carry):
                for b in range(NB):
                    step(i * NB + b, b)
                return carry

            lax.fori_loop(1, CHP // NB - 1, outer, 0)
            for b in range(NB):
                step(CHP - NB + b, b, fire=(b == 0))
            wait_scatter((CHP - 1) % NB)
            dwait()
        plsc.subcore_barrier()

        pltpu.sync_copy(agg_sh.at[pl.ds(sid * ROWS, ROWS)],
                        agg_out.at[cid, pl.ds(sid * ROWS, ROWS)])
        if with_deg:
            @pl.when(sid == 0)
            def _():
                pltpu.sync_copy(deg_sh, deg_out.at[cid])

    k = pl.kernel(
        body,
        out_type=tuple(out_type) if with_deg else out_type[0],
        mesh=plsc.VectorSubcoreMesh(**_MESH),
        scratch_types=scratch,
    )
    return k(h, ei_r, zrows, zdeg)


def kernel(x, edge_index, batch, W_fc1, b_fc1, W_gc1, b_gc1, W_gc2, b_gc2,
           W_fc2, b_fc2, W_fc3, b_fc3):
    pad = EP - E
    r = jnp.arange(pad, dtype=jnp.int32)
    ei_pad = jnp.concatenate(
        [edge_index,
         jnp.stack([r % N, N + r % (NP_ - N)])],
        axis=1)
    ei_r = ei_pad.reshape(2, NW, CH, K)
    zrows = jnp.zeros((ROWS, D), jnp.float32)
    zdeg = jnp.zeros((NP_,), jnp.float32)

    agg1, degp = _sc_agg(x, ei_r, zrows, zdeg, with_deg=True)
    degp = degp.T
    h1 = _conv_linear(agg1, degp, W_fc1, b_fc1, W_gc1, b_gc1)
    agg2, _ = _sc_agg(h1, ei_r, zrows, zdeg, with_deg=True)
    return _tail(agg2, degp, batch, W_gc2, b_gc2, W_fc2, b_fc2, W_fc3, b_fc3)


# speedup vs baseline: 1.0439x; 1.0439x over previous
import jax
import jax.numpy as jnp
from jax import lax
from jax.experimental import pallas as pl
from jax.experimental.pallas import tpu as pltpu
from jax.experimental.pallas import tpu_sc as plsc

N = 10000
E = 320000
D = 128
G = 64

NC = 2
NS = 16
NW = NC * NS
K = 128
CH = 80
NPASS = 5
CHP = CH // NPASS
EPW = CH * K
EP = NW * EPW
NB = 2
NP_ = 10240
ROWS = NP_ // NS

BN = 1000
GRID = N // BN


def _leaky(x):
    return jnp.where(x > 0, x, 0.01 * x)


def _fc1_body(x_ref, w_ref, b_ref, o_ref):
    o_ref[...] = lax.dot_general(
        x_ref[...], w_ref[...], (((1,), (1,)), ((), ())),
        preferred_element_type=jnp.float32) + b_ref[...]


def _fc1(x, W, b):
    return pl.pallas_call(
        _fc1_body,
        grid=(GRID,),
        in_specs=[
            pl.BlockSpec((BN, D), lambda i: (i, 0)),
            pl.BlockSpec((D, D), lambda i: (0, 0)),
            pl.BlockSpec((1, D), lambda i: (0, 0)),
        ],
        out_specs=pl.BlockSpec((BN, D), lambda i: (i, 0)),
        out_shape=jax.ShapeDtypeStruct((N, D), jnp.float32),
    )(x, W, b.reshape(1, D))


def _conv_linear_body(a_ref, d_ref, w1_ref, b1_ref, w_ref, b_ref, o_ref):
    a = a_ref[0] + a_ref[1]
    deg = jnp.clip(d_ref[...][:, 0] + d_ref[...][:, 1], 1.0, None)
    m = a / deg[:, None]
    m = lax.dot_general(
        m, w1_ref[...], (((1,), (1,)), ((), ())),
        preferred_element_type=jnp.float32) + b1_ref[...]
    o_ref[...] = _leaky(lax.dot_general(
        m, w_ref[...], (((1,), (1,)), ((), ())),
        preferred_element_type=jnp.float32) + b_ref[...])


def _conv_linear(aggp, degp, W1, b1, W, b):
    return pl.pallas_call(
        _conv_linear_body,
        grid=(GRID,),
        in_specs=[
            pl.BlockSpec((NC, BN, D), lambda i: (0, i, 0)),
            pl.BlockSpec((BN, NC), lambda i: (i, 0)),
            pl.BlockSpec((D, D), lambda i: (0, 0)),
            pl.BlockSpec((1, D), lambda i: (0, 0)),
            pl.BlockSpec((D, D), lambda i: (0, 0)),
            pl.BlockSpec((1, D), lambda i: (0, 0)),
        ],
        out_specs=pl.BlockSpec((BN, D), lambda i: (i, 0)),
        out_shape=jax.ShapeDtypeStruct((N, D), jnp.float32),
    )(aggp, degp, W1, b1.reshape(1, D), W, b.reshape(1, D))


def _tail_body(a_ref, d_ref, bat_ref, wg_ref, bg_ref, w2_ref, b2_ref,
               w3_ref, b3_ref, o_ref, sums_sc, cnt_sc):
    i = pl.program_id(0)

    @pl.when(i == 0)
    def _():
        sums_sc[...] = jnp.zeros_like(sums_sc)
        cnt_sc[...] = jnp.zeros_like(cnt_sc)

    a = a_ref[0] + a_ref[1]
    deg = jnp.clip(d_ref[...][:, 0] + d_ref[...][:, 1], 1.0, None)
    m = a / deg[:, None]
    h2 = _leaky(lax.dot_general(
        m, wg_ref[...], (((1,), (1,)), ((), ())),
        preferred_element_type=jnp.float32) + bg_ref[...])

    bvec = bat_ref[...][:, 0]
    onehot = (bvec[:, None] ==
              lax.broadcasted_iota(jnp.int32, (1, G), 1)).astype(jnp.float32)
    sums_sc[...] += lax.dot_general(
        onehot, h2, (((0,), (0,)), ((), ())),
        preferred_element_type=jnp.float32)
    cnt_sc[...] += jnp.sum(onehot, axis=0)[:, None]

    @pl.when(i == GRID - 1)
    def _():
        cnt = jnp.clip(cnt_sc[...][:, 0:1], 1.0, None)
        hg = sums_sc[...] / cnt
        nrm = jnp.clip(
            jnp.sqrt(jnp.sum(hg * hg, axis=1, keepdims=True)), 1e-12, None)
        hg = hg / nrm
        h = _leaky(lax.dot_general(
            hg, w2_ref[...], (((1,), (1,)), ((), ())),
            preferred_element_type=jnp.float32) + b2_ref[...])
        o_ref[...] = lax.dot_general(
            h, w3_ref[...], (((1,), (1,)), ((), ())),
            preferred_element_type=jnp.float32) + b3_ref[...]


def _tail(aggp, degp, batch, Wg, bg, W2, b2, W3, b3):
    return pl.pallas_call(
        _tail_body,
        grid=(GRID,),
        in_specs=[
            pl.BlockSpec((NC, BN, D), lambda i: (0, i, 0)),
            pl.BlockSpec((BN, NC), lambda i: (i, 0)),
            pl.BlockSpec((BN, 1), lambda i: (i, 0)),
            pl.BlockSpec((D, D), lambda i: (0, 0)),
            pl.BlockSpec((1, D), lambda i: (0, 0)),
            pl.BlockSpec((D, D), lambda i: (0, 0)),
            pl.BlockSpec((1, D), lambda i: (0, 0)),
            pl.BlockSpec((D, D), lambda i: (0, 0)),
            pl.BlockSpec((1, D), lambda i: (0, 0)),
        ],
        out_specs=pl.BlockSpec((G, D), lambda i: (0, 0)),
        out_shape=jax.ShapeDtypeStruct((G, D), jnp.float32),
        scratch_shapes=[
            pltpu.VMEM((G, D), jnp.float32),
            pltpu.VMEM((G, D), jnp.float32),
        ],
    )(aggp, degp, batch.reshape(N, 1), Wg, bg.reshape(1, D),
      W2, b2.reshape(1, D), W3, b3.reshape(1, D))


_MESH = dict(core_axis_name="c", subcore_axis_name="s",
             num_cores=NC, num_subcores=NS)


def _sc_agg(h, ei_r, zrows, zdeg, with_deg):
    out_type = [jax.ShapeDtypeStruct((NC, NP_, D), jnp.float32)]
    scratch = (
        [pltpu.VMEM((CHP, K), jnp.int32),
         pltpu.VMEM((CHP, K), jnp.int32)]
        + [pltpu.VMEM((K, D), jnp.float32)] * NB
        + [pltpu.VMEM_SHARED((NP_, D), jnp.float32)]
        + [pltpu.SemaphoreType.DMA] * (2 * NB)
    )
    if with_deg:
        out_type.append(jax.ShapeDtypeStruct((NC, NP_), jnp.float32))
        scratch += [
            pltpu.VMEM((K,), jnp.float32),
            pltpu.VMEM_SHARED((NP_,), jnp.float32),
            pltpu.SemaphoreType.DMA,
        ]

    def body(h_hbm, ei_hbm, zr_hbm, zd_hbm, *rest):
        agg_out = rest[0]
        rest = rest[1:]
        if with_deg:
            deg_out = rest[0]
            rest = rest[1:]
        src_v, dst_v = rest[0], rest[1]
        msgs = rest[2:2 + NB]
        agg_sh = rest[2 + NB]
        sem_g = rest[3 + NB:3 + 2 * NB]
        sem_s = rest[3 + 2 * NB:3 + 3 * NB]
        if with_deg:
            ones_v, deg_sh, sem_d = rest[3 + 3 * NB:]
        cid = lax.axis_index("c")
        sid = lax.axis_index("s")
        wid = cid * NS + sid

        pltpu.sync_copy(zr_hbm, agg_sh.at[pl.ds(sid * ROWS, ROWS)])
        if with_deg:
            @pl.when(sid == 0)
            def _():
                pltpu.sync_copy(zd_hbm, deg_sh)
            for j in range(K // 16):
                ones_v[pl.ds(j * 16, 16)] = jnp.ones((16,), jnp.float32)

        plsc.subcore_barrier()

        def wait_gather(b):
            pltpu.make_async_copy(h_hbm.at[pl.ds(0, K)], msgs[b],
                                  sem_g[b]).wait()

        def wait_scatter(b):
            pltpu.make_async_copy(h_hbm.at[pl.ds(0, K)], msgs[b],
                                  sem_s[b]).wait()

        def wait_deg():
            pltpu.make_async_copy(zd_hbm.at[pl.ds(0, K)], ones_v,
                                  sem_d).wait()

        def fire_gather(c, b):
            pltpu.async_copy(h_hbm.at[src_v.at[c]], msgs[b], sem_g[b])

        def fire_scatter(c, b):
            pltpu.async_copy(msgs[b], agg_sh.at[dst_v.at[c]], sem_s[b],
                             add=True)

        def fire_deg(c):
            if with_deg:
                pltpu.async_copy(ones_v, deg_sh.at[dst_v.at[c]], sem_d,
                                 add=True)

        def dwait():
            if with_deg:
                wait_deg()

        def step(c, b, first=False, last=False):
            if not first:
                wait_scatter(b ^ 1)
                dwait()
            if not last:
                fire_gather(c + 1, b ^ 1)
            wait_gather(b)
            fire_scatter(c, b)
            fire_deg(c)

        for p in range(NPASS):
            pltpu.sync_copy(ei_hbm.at[0, wid, pl.ds(p * CHP, CHP)], src_v)
            pltpu.sync_copy(ei_hbm.at[1, wid, pl.ds(p * CHP, CHP)], dst_v)
            fire_gather(0, 0)
            step(0, 0, first=True)
            step(1, 1)

            def outer(i, carry):
                step(i * 2, 0)
                step(i * 2 + 1, 1)
                return carry

            lax.fori_loop(1, CHP // 2 - 1, outer, 0)
            step(CHP - 2, 0)
            step(CHP - 1, 1, last=True)
            wait_scatter(1)
            dwait()
        plsc.subcore_barrier()

        pltpu.sync_copy(agg_sh.at[pl.ds(sid * ROWS, ROWS)],
                        agg_out.at[cid, pl.ds(sid * ROWS, ROWS)])
        if with_deg:
            @pl.when(sid == 0)
            def _():
                pltpu.sync_copy(deg_sh, deg_out.at[cid])

    k = pl.kernel(
        body,
        out_type=tuple(out_type) if with_deg else out_type[0],
        mesh=plsc.VectorSubcoreMesh(**_MESH),
        scratch_types=scratch,
    )
    return k(h, ei_r, zrows, zdeg)


def kernel(x, edge_index, batch, W_fc1, b_fc1, W_gc1, b_gc1, W_gc2, b_gc2,
           W_fc2, b_fc2, W_fc3, b_fc3):
    pad = EP - E
    r = jnp.arange(pad, dtype=jnp.int32)
    ei_pad = jnp.concatenate(
        [edge_index,
         jnp.stack([r % N, N + r % (NP_ - N)])],
        axis=1)
    ei_r = ei_pad.reshape(2, NW, CH, K)
    zrows = jnp.zeros((ROWS, D), jnp.float32)
    zdeg = jnp.zeros((NP_,), jnp.float32)

    agg1, degp = _sc_agg(x, ei_r, zrows, zdeg, with_deg=True)
    degp = degp.T
    h1 = _conv_linear(agg1, degp, W_fc1, b_fc1, W_gc1, b_gc1)
    agg2, _ = _sc_agg(h1, ei_r, zrows, zdeg, with_deg=True)
    return _tail(agg2, degp, batch, W_gc2, b_gc2, W_fc2, b_fc2, W_fc3, b_fc3)
